# baseline (speedup 1.0000x reference)
import functools

import jax
import jax.numpy as jnp
from jax import lax
from jax.experimental import pallas as pl
from jax.experimental.pallas import tpu as pltpu
from jax.experimental.pallas import tpu_sc as plsc

_B, _H, _W = 16, 512, 512
_N = _B * _H * _W
_ROWS_PER_WORKER = _H // 2
_CHUNKS = _W // 16


def _bcast16(vec, k):
    idx = jnp.broadcast_to(jnp.asarray(k, jnp.int32).reshape(1, 1), (16, 1))
    return lax.gather(
        vec, idx,
        lax.GatherDimensionNumbers(offset_dims=(), collapsed_slice_dims=(0,),
                                   start_index_map=(0,)),
        slice_sizes=(1,),
        mode=lax.GatherScatterMode.PROMISE_IN_BOUNDS)


def _bf16_round(x):
    u = lax.bitcast_convert_type(x, jnp.int32)
    u = u + 0x7FFF + ((u >> 16) & 1)
    return lax.bitcast_convert_type(u & jnp.int32(-65536), jnp.float32)


def _sc_body(theta_hbm, grid_hbm, img_hbm, out_hbm,
             theta_v, grid_v, i00, i01, i10, i11, w00, w01, w10, w11,
             v00, v01, v10, v11, out_v, sem, sem_o):
    wid = lax.axis_index("s") * 2 + lax.axis_index("c")
    b = wid // 2
    y0 = (wid % 2) * _ROWS_PER_WORKER
    bbase = b * (_H * _W)

    pltpu.async_copy(theta_hbm, theta_v, sem_o).wait()
    pltpu.async_copy(grid_hbm, grid_v, sem_o).wait()
    trow = theta_v[pl.ds(b * 16, 16)]
    t0, t1, t2 = _bcast16(trow, 0), _bcast16(trow, 1), _bcast16(trow, 2)
    t3, t4, t5 = _bcast16(trow, 3), _bcast16(trow, 4), _bcast16(trow, 5)

    def row_body(i, _):
        y = y0 + i
        ywin = grid_v[pl.ds(y & ~jnp.int32(15), 16)]
        yb = _bcast16(ywin, y & 15)
        cx_row = t1 * yb + t2
        cy_row = t4 * yb + t5

        def chunk_body(j, _):
            xb = grid_v[pl.ds(j * 16, 16)]
            sx = t0 * xb + cx_row
            sy = t3 * xb + cy_row
            rx = _bf16_round(sx) * 256.0 + 256.0
            ry = _bf16_round(sy) * 256.0 + 256.0
            fx = rx.astype(jnp.int32)
            fx = jnp.where(fx.astype(jnp.float32) > rx, fx - 1, fx)
            fy = ry.astype(jnp.int32)
            fy = jnp.where(fy.astype(jnp.float32) > ry, fy - 1, fy)
            cx0 = jnp.clip(fx, 0, _W - 1)
            cx1 = jnp.clip(fx + 1, 0, _W - 1)
            cy0 = jnp.clip(fy, 0, _H - 1)
            cy1 = jnp.clip(fy + 1, 0, _H - 1)
            wx0 = jnp.maximum(0.0, 1.0 - jnp.abs(rx - cx0.astype(jnp.float32)))
            wx1 = jnp.maximum(0.0, 1.0 - jnp.abs(rx - cx1.astype(jnp.float32)))
            wy0 = jnp.maximum(0.0, 1.0 - jnp.abs(ry - cy0.astype(jnp.float32)))
            wy1 = jnp.maximum(0.0, 1.0 - jnp.abs(ry - cy1.astype(jnp.float32)))
            r0 = bbase + cy0 * _W
            r1 = bbase + cy1 * _W
            wsl = pl.ds(j * 16, 16)
            i00[wsl] = r0 + cx0
            i01[wsl] = r0 + cx1
            i10[wsl] = r1 + cx0
            i11[wsl] = r1 + cx1
            w00[wsl] = wx0 * wy0
            w01[wsl] = wx1 * wy0
            w10[wsl] = wx0 * wy1
            w11[wsl] = wx1 * wy1
            return _

        lax.fori_loop(0, _CHUNKS, chunk_body, None)

        copies = [pltpu.async_copy(img_hbm.at[i00], v00, sem),
                  pltpu.async_copy(img_hbm.at[i01], v01, sem),
                  pltpu.async_copy(img_hbm.at[i10], v10, sem),
                  pltpu.async_copy(img_hbm.at[i11], v11, sem)]
        for cp in copies:
            cp.wait()

        def interp_body(j, _):
            wsl = pl.ds(j * 16, 16)
            acc = (v00[wsl] * w00[wsl] + v01[wsl] * w01[wsl] +
                   v10[wsl] * w10[wsl] + v11[wsl] * w11[wsl])
            out_v[wsl] = jnp.clip(acc, 0.0, 1.0)
            return _

        lax.fori_loop(0, _CHUNKS, interp_body, None)
        pltpu.async_copy(out_v, out_hbm.at[pl.ds(bbase + y * _W, _W)], sem_o).wait()
        return _

    lax.fori_loop(0, _ROWS_PER_WORKER, row_body, None)


def kernel(theta, image):
    theta_b = theta.astype(jnp.bfloat16).astype(jnp.float32)
    theta_flat = jnp.pad(theta_b, ((0, 0), (0, 10))).reshape(-1)
    grid = jnp.linspace(-1.0, 1.0, _W).astype(jnp.bfloat16).astype(jnp.float32)
    img_flat = image.reshape(-1)
    mesh = plsc.VectorSubcoreMesh(core_axis_name="c", subcore_axis_name="s")
    run = functools.partial(
        pl.kernel,
        mesh=mesh,
        out_type=jax.ShapeDtypeStruct((_N,), jnp.float32),
        scratch_types=[
            pltpu.VMEM((_B * 16,), jnp.float32),
            pltpu.VMEM((_W,), jnp.float32),
            pltpu.VMEM((_W,), jnp.int32),
            pltpu.VMEM((_W,), jnp.int32),
            pltpu.VMEM((_W,), jnp.int32),
            pltpu.VMEM((_W,), jnp.int32),
            pltpu.VMEM((_W,), jnp.float32),
            pltpu.VMEM((_W,), jnp.float32),
            pltpu.VMEM((_W,), jnp.float32),
            pltpu.VMEM((_W,), jnp.float32),
            pltpu.VMEM((_W,), jnp.float32),
            pltpu.VMEM((_W,), jnp.float32),
            pltpu.VMEM((_W,), jnp.float32),
            pltpu.VMEM((_W,), jnp.float32),
            pltpu.VMEM((_W,), jnp.float32),
            pltpu.SemaphoreType.DMA,
            pltpu.SemaphoreType.DMA,
        ],
    )(_sc_body)
    out = run(theta_flat, grid, img_flat)
    return out.reshape(image.shape)


# speedup vs baseline: 1.0209x; 1.0209x over previous
import functools

import jax
import jax.numpy as jnp
from jax import lax
from jax.experimental import pallas as pl
from jax.experimental.pallas import tpu as pltpu
from jax.experimental.pallas import tpu_sc as plsc

_B, _H, _W = 16, 512, 512
_N = _B * _H * _W
_ROWS_PER_WORKER = _H // 2
_CHUNKS = _W // 16


def _bcast16(vec, k):
    idx = jnp.broadcast_to(jnp.asarray(k, jnp.int32).reshape(1, 1), (16, 1))
    return lax.gather(
        vec, idx,
        lax.GatherDimensionNumbers(offset_dims=(), collapsed_slice_dims=(0,),
                                   start_index_map=(0,)),
        slice_sizes=(1,),
        mode=lax.GatherScatterMode.PROMISE_IN_BOUNDS)


def _bf16_round(x):
    u = lax.bitcast_convert_type(x, jnp.int32)
    u = u + 0x7FFF + ((u >> 16) & 1)
    return lax.bitcast_convert_type(u & jnp.int32(-65536), jnp.float32)


def _sc_body(theta_hbm, grid_hbm, img_hbm, out_hbm,
             theta_v, grid_v, i00, i01, i10, i11, w00, w01, w10, w11,
             v00, v01, v10, v11, out_v,
             j00, j01, j10, j11, x00, x01, x10, x11,
             u00, u01, u10, u11, out_u, sem, sem_o):
    wid = lax.axis_index("s") * 2 + lax.axis_index("c")
    b = wid // 2
    y0 = (wid % 2) * _ROWS_PER_WORKER
    bbase = b * (_H * _W)

    pltpu.async_copy(theta_hbm, theta_v, sem_o).wait()
    pltpu.async_copy(grid_hbm, grid_v, sem_o).wait()
    trow = theta_v[pl.ds(b * 16, 16)]
    t0, t1, t2 = _bcast16(trow, 0), _bcast16(trow, 1), _bcast16(trow, 2)
    t3, t4, t5 = _bcast16(trow, 3), _bcast16(trow, 4), _bcast16(trow, 5)

    def compute_row(y, bufs):
        iA, iB, iC, iD, wA, wB, wC, wD = bufs[:8]
        ywin = grid_v[pl.ds(y & ~jnp.int32(15), 16)]
        yb = _bcast16(ywin, y & 15)
        cx_row = t1 * yb + t2
        cy_row = t4 * yb + t5

        def chunk_body(j, _):
            xb = grid_v[pl.ds(j * 16, 16)]
            sx = t0 * xb + cx_row
            sy = t3 * xb + cy_row
            rx = _bf16_round(sx) * 256.0 + 256.0
            ry = _bf16_round(sy) * 256.0 + 256.0
            fx = rx.astype(jnp.int32)
            fx = jnp.where(fx.astype(jnp.float32) > rx, fx - 1, fx)
            fy = ry.astype(jnp.int32)
            fy = jnp.where(fy.astype(jnp.float32) > ry, fy - 1, fy)
            cx0 = jnp.clip(fx, 0, _W - 1)
            cx1 = jnp.clip(fx + 1, 0, _W - 1)
            cy0 = jnp.clip(fy, 0, _H - 1)
            cy1 = jnp.clip(fy + 1, 0, _H - 1)
            wx0 = jnp.maximum(0.0, 1.0 - jnp.abs(rx - cx0.astype(jnp.float32)))
            wx1 = jnp.maximum(0.0, 1.0 - jnp.abs(rx - cx1.astype(jnp.float32)))
            wy0 = jnp.maximum(0.0, 1.0 - jnp.abs(ry - cy0.astype(jnp.float32)))
            wy1 = jnp.maximum(0.0, 1.0 - jnp.abs(ry - cy1.astype(jnp.float32)))
            r0 = bbase + cy0 * _W
            r1 = bbase + cy1 * _W
            wsl = pl.ds(j * 16, 16)
            iA[wsl] = r0 + cx0
            iB[wsl] = r0 + cx1
            iC[wsl] = r1 + cx0
            iD[wsl] = r1 + cx1
            wA[wsl] = wx0 * wy0
            wB[wsl] = wx1 * wy0
            wC[wsl] = wx0 * wy1
            wD[wsl] = wx1 * wy1
            return _

        lax.fori_loop(0, _CHUNKS, chunk_body, None)

    def fire(bufs):
        iA, iB, iC, iD = bufs[:4]
        vA, vB, vC, vD = bufs[8:12]
        pltpu.async_copy(img_hbm.at[iA], vA, sem)
        pltpu.async_copy(img_hbm.at[iB], vB, sem)
        pltpu.async_copy(img_hbm.at[iC], vC, sem)
        pltpu.async_copy(img_hbm.at[iD], vD, sem)

    def drain(bufs):
        iA, iB, iC, iD = bufs[:4]
        vA, vB, vC, vD = bufs[8:12]
        pltpu.make_async_copy(img_hbm.at[iA], vA, sem).wait()
        pltpu.make_async_copy(img_hbm.at[iB], vB, sem).wait()
        pltpu.make_async_copy(img_hbm.at[iC], vC, sem).wait()
        pltpu.make_async_copy(img_hbm.at[iD], vD, sem).wait()

    def interp_out(y, bufs):
        wA, wB, wC, wD = bufs[4:8]
        vA, vB, vC, vD = bufs[8:12]
        ov = bufs[12]

        def interp_body(j, _):
            wsl = pl.ds(j * 16, 16)
            acc = (vA[wsl] * wA[wsl] + vB[wsl] * wB[wsl] +
                   vC[wsl] * wC[wsl] + vD[wsl] * wD[wsl])
            ov[wsl] = jnp.clip(acc, 0.0, 1.0)
            return _

        lax.fori_loop(0, _CHUNKS, interp_body, None)
        pltpu.async_copy(ov, out_hbm.at[pl.ds(bbase + y * _W, _W)], sem_o)

    def drain_out(bufs):
        pltpu.make_async_copy(bufs[12], out_hbm.at[pl.ds(0, _W)], sem_o).wait()

    bufsA = (i00, i01, i10, i11, w00, w01, w10, w11, v00, v01, v10, v11, out_v)
    bufsB = (j00, j01, j10, j11, x00, x01, x10, x11, u00, u01, u10, u11, out_u)

    compute_row(y0, bufsA)
    fire(bufsA)

    def pair_body(g, _):
        r0 = y0 + 2 * g
        compute_row(r0 + 1, bufsB)
        fire(bufsB)
        drain(bufsA)

        @pl.when(g > 0)
        def _():
            drain_out(bufsA)

        interp_out(r0, bufsA)

        @pl.when(g < _ROWS_PER_WORKER // 2 - 1)
        def _():
            compute_row(r0 + 2, bufsA)
            fire(bufsA)

        drain(bufsB)

        @pl.when(g > 0)
        def _():
            drain_out(bufsB)

        interp_out(r0 + 1, bufsB)
        return _

    lax.fori_loop(0, _ROWS_PER_WORKER // 2, pair_body, None)
    drain_out(bufsA)
    drain_out(bufsB)


def kernel(theta, image):
    theta_b = theta.astype(jnp.bfloat16).astype(jnp.float32)
    theta_flat = jnp.pad(theta_b, ((0, 0), (0, 10))).reshape(-1)
    grid = jnp.linspace(-1.0, 1.0, _W).astype(jnp.bfloat16).astype(jnp.float32)
    img_flat = image.reshape(-1)
    mesh = plsc.VectorSubcoreMesh(core_axis_name="c", subcore_axis_name="s")
    run = functools.partial(
        pl.kernel,
        mesh=mesh,
        out_type=jax.ShapeDtypeStruct((_N,), jnp.float32),
        scratch_types=[
            pltpu.VMEM((_B * 16,), jnp.float32),
            pltpu.VMEM((_W,), jnp.float32),
            pltpu.VMEM((_W,), jnp.int32),
            pltpu.VMEM((_W,), jnp.int32),
            pltpu.VMEM((_W,), jnp.int32),
            pltpu.VMEM((_W,), jnp.int32),
            pltpu.VMEM((_W,), jnp.float32),
            pltpu.VMEM((_W,), jnp.float32),
            pltpu.VMEM((_W,), jnp.float32),
            pltpu.VMEM((_W,), jnp.float32),
            pltpu.VMEM((_W,), jnp.float32),
            pltpu.VMEM((_W,), jnp.float32),
            pltpu.VMEM((_W,), jnp.float32),
            pltpu.VMEM((_W,), jnp.float32),
            pltpu.VMEM((_W,), jnp.float32),
            pltpu.VMEM((_W,), jnp.int32),
            pltpu.VMEM((_W,), jnp.int32),
            pltpu.VMEM((_W,), jnp.int32),
            pltpu.VMEM((_W,), jnp.int32),
            pltpu.VMEM((_W,), jnp.float32),
            pltpu.VMEM((_W,), jnp.float32),
            pltpu.VMEM((_W,), jnp.float32),
            pltpu.VMEM((_W,), jnp.float32),
            pltpu.VMEM((_W,), jnp.float32),
            pltpu.VMEM((_W,), jnp.float32),
            pltpu.VMEM((_W,), jnp.float32),
            pltpu.VMEM((_W,), jnp.float32),
            pltpu.VMEM((_W,), jnp.float32),
            pltpu.SemaphoreType.DMA,
            pltpu.SemaphoreType.DMA,
        ],
    )(_sc_body)
    out = run(theta_flat, grid, img_flat)
    return out.reshape(image.shape)
